# baseline (speedup 1.0000x reference)
import jax
import jax.numpy as jnp
from jax.experimental import pallas as pl
from jax.experimental.pallas import tpu as pltpu

_KC = 200


def _qtab_kernel(in_ref, mask_ref, table_ref, out_ref):
    k = pl.program_id(0)
    acc = jnp.dot(
        table_ref[0].astype(jnp.bfloat16),
        in_ref[...].astype(jnp.bfloat16),
        preferred_element_type=jnp.float32,
    )

    @pl.when(k == 0)
    def _first():
        out_ref[...] = acc + mask_ref[...]

    @pl.when(k > 0)
    def _rest():
        out_ref[...] = out_ref[...] + acc


def kernel(inputs, mask, table):
    B, K = inputs.shape
    N = table.shape[1]
    nk = K // _KC
    table_chunks = table.T.reshape(N, nk, _KC).swapaxes(0, 1)
    out_t = pl.pallas_call(
        _qtab_kernel,
        grid=(nk,),
        in_specs=[
            pl.BlockSpec((_KC, B), lambda k: (k, 0)),
            pl.BlockSpec((N, B), lambda k: (0, 0)),
            pl.BlockSpec((1, N, _KC), lambda k: (k, 0, 0)),
        ],
        out_specs=pl.BlockSpec((N, B), lambda k: (0, 0)),
        out_shape=jax.ShapeDtypeStruct((N, B), jnp.float32),
        compiler_params=pltpu.CompilerParams(
            dimension_semantics=("arbitrary",),
        ),
    )(inputs.T, mask.T, table_chunks)
    return out_t.T
